# baseline (speedup 1.0000x reference)
import jax
import jax.numpy as jnp
from jax import lax
from jax.experimental import pallas as pl
from jax.experimental.pallas import tpu as pltpu
from jax.experimental.pallas import tpu_sc as plsc

N = 20000
D = 512
N_INS = 8
LANES = 16
NTILES = 16
CHUNK = 1280
LAST_BASE = N - CHUNK
LAST_LO = (NTILES - 1) * CHUNK
INNER = 4
OUTER = CHUNK // (LANES * INNER)
BIG_I = 2 ** 30
SENT = -2.0
MASKED = -3.0


def _insert_step(v, gi, ms, mi):
    c = [v > m for m in ms]
    nm = [jnp.where(c[0], v, ms[0])]
    ni = [jnp.where(c[0], gi, mi[0])]
    for k in range(1, N_INS):
        nm.append(jnp.where(c[k], jnp.where(c[k - 1], ms[k - 1], v), ms[k]))
        ni.append(jnp.where(c[k], jnp.where(c[k - 1], mi[k - 1], gi), mi[k]))
    return nm, ni


def _extract8(ms, mi, iota):
    def round_body(r, st):
        selv, seli, pv, pi = st
        m = jnp.full((LANES,), SENT, jnp.float32)
        ii = jnp.full((LANES,), BIG_I, jnp.int32)
        for k in range(N_INS):
            v, gi = ms[k], mi[k]
            elig = (v < pv) | ((v == pv) & (gi > pi))
            veff = jnp.where(elig, v, jnp.float32(SENT))
            upd = veff > m
            m = jnp.where(upd, veff, m)
            ii = jnp.where(upd, gi, ii)
        mval = jnp.max(m)
        midx = jnp.min(jnp.where(m == mval, ii, BIG_I))
        selv = jnp.where(iota == r, mval, selv)
        seli = jnp.where(iota == r, midx, seli)
        return (selv, seli, mval, midx)

    st0 = (jnp.full((LANES,), SENT, jnp.float32),
           jnp.zeros((LANES,), jnp.int32),
           jnp.float32(2.0), jnp.int32(-1))
    selv, seli, _, _ = lax.fori_loop(0, N_INS, round_body, st0)
    return selv, seli


def _fresh_state():
    return ([jnp.full((LANES,), SENT, jnp.float32) for _ in range(N_INS)],
            [jnp.full((LANES,), BIG_I, jnp.int32) for _ in range(N_INS)])


WB_OFF = N
WB_LEN = 8 + 2 * D
W_OFF = 8


def _sc_body(pk_hbm, h_hbm, probs_hbm,
             a_v, st_f, st_i, spm_f, spm_i, cand_f, cand_i,
             idx_v, rows_v, wb_v, o0_v, o1_v, sem, sem2):
    cid = lax.axis_index("c")
    sid = lax.axis_index("s")
    iota = lax.iota(jnp.int32, LANES)
    last = sid == NTILES - 1
    base = jnp.where(last, LAST_BASE, sid * CHUNK)
    lo = jnp.where(last, LAST_LO, 0)

    copy_a = pltpu.async_copy(pk_hbm.at[pl.ds(base, CHUNK)], a_v, sem)

    @pl.when(sid == 0)
    def _():
        pltpu.async_copy(pk_hbm.at[pl.ds(WB_OFF, WB_LEN)], wb_v, sem2)

    copy_a.wait()

    sgn = jnp.where(cid == 0, jnp.float32(1.0), jnp.float32(-1.0))
    bi = base + iota

    def scan_body(j, st):
        ms, mi = list(st[0]), list(st[1])
        off0 = j * (LANES * INNER)
        for k in range(INNER):
            off = off0 + k * LANES
            gi = bi + off
            v = a_v[pl.ds(off, LANES)] * sgn
            v = jnp.where(gi >= lo, v, jnp.float32(MASKED))
            ms, mi = _insert_step(v, gi, ms, mi)
        return (tuple(ms), tuple(mi))

    ms0, mi0 = _fresh_state()
    ms, mi = lax.fori_loop(0, OUTER, scan_body, (tuple(ms0), tuple(mi0)))
    selv, seli = _extract8(list(ms), list(mi), iota)

    st_f[...] = selv
    st_i[...] = seli
    pltpu.sync_copy(st_f, spm_f.at[pl.ds(sid * LANES, LANES)])
    pltpu.sync_copy(st_i, spm_i.at[pl.ds(sid * LANES, LANES)])
    plsc.subcore_barrier()

    @pl.when(sid == 0)
    def _():
        pltpu.sync_copy(spm_f, cand_f)
        pltpu.sync_copy(spm_i, cand_i)

        def merge_body(t, st):
            ms, mi = list(st[0]), list(st[1])
            v = cand_f[pl.ds(t * LANES, LANES)]
            gi = cand_i[pl.ds(t * LANES, LANES)]
            ms, mi = _insert_step(v, gi, ms, mi)
            return (tuple(ms), tuple(mi))

        ms0, mi0 = _fresh_state()
        ms, mi = lax.fori_loop(0, NTILES, merge_body, (tuple(ms0), tuple(mi0)))
        _, gsel = _extract8(list(ms), list(mi), iota)

        idx_v[...] = jnp.where(iota < N_INS, gsel, 0)
        pltpu.async_copy(h_hbm.at[idx_v], rows_v, sem).wait()

        pltpu.make_async_copy(pk_hbm.at[pl.ds(WB_OFF, WB_LEN)], wb_v,
                              sem2).wait()

        def mm_body(j, st):
            a0s, a1s = list(st[0]), list(st[1])
            wt0 = wb_v[pl.ds(W_OFF + j * LANES, LANES)]
            wt1 = wb_v[pl.ds(W_OFF + D + j * LANES, LANES)]
            for i in range(N_INS):
                rv = rows_v[i, 0, pl.ds(j * LANES, LANES)]
                a0s[i] = a0s[i] + rv * wt0
                a1s[i] = a1s[i] + rv * wt1
            return (tuple(a0s), tuple(a1s))

        z = [jnp.zeros((LANES,), jnp.float32) for _ in range(N_INS)]
        a0s, a1s = lax.fori_loop(0, D // LANES, mm_body,
                                 (tuple(z), tuple(z)))
        l0 = jnp.zeros((LANES,), jnp.float32)
        l1 = jnp.zeros((LANES,), jnp.float32)
        for i in range(N_INS):
            l0 = jnp.where(iota == i, jnp.sum(a0s[i]), l0)
            l1 = jnp.where(iota == i, jnp.sum(a1s[i]), l1)

        bv = wb_v[pl.ds(0, LANES)]
        l0 = l0 + bv[0]
        l1 = l1 + bv[1]
        o0_v[...] = 1.0 / (1.0 + jnp.exp(l1 - l0))
        o1_v[...] = 1.0 / (1.0 + jnp.exp(l0 - l1))
        pltpu.sync_copy(o0_v.at[pl.ds(0, N_INS)],
                        probs_hbm.at[pl.ds(cid * N_INS, N_INS)])
        pltpu.sync_copy(o1_v.at[pl.ds(0, N_INS)],
                        probs_hbm.at[pl.ds(2 * N_INS + cid * N_INS, N_INS)])


@jax.jit
def _sc_call(packed, h):
    mesh = plsc.VectorSubcoreMesh(core_axis_name="c", subcore_axis_name="s")
    fn = pl.kernel(
        _sc_body,
        mesh=mesh,
        out_type=jax.ShapeDtypeStruct((4 * N_INS,), jnp.float32),
        compiler_params=pltpu.CompilerParams(
            needs_layout_passes=False, use_tc_tiling_on_sc=False,
            disable_bounds_checks=True, disable_semaphore_checks=True),
        scratch_types=[
            pltpu.VMEM((CHUNK,), jnp.float32),
            pltpu.VMEM((LANES,), jnp.float32),
            pltpu.VMEM((LANES,), jnp.int32),
            pltpu.VMEM_SHARED((NTILES * LANES,), jnp.float32),
            pltpu.VMEM_SHARED((NTILES * LANES,), jnp.int32),
            pltpu.VMEM((NTILES * LANES,), jnp.float32),
            pltpu.VMEM((NTILES * LANES,), jnp.int32),
            pltpu.VMEM((LANES,), jnp.int32),
            pltpu.VMEM((LANES, 1, D), jnp.float32),
            pltpu.VMEM((8 + 2 * D,), jnp.float32),
            pltpu.VMEM((LANES,), jnp.float32),
            pltpu.VMEM((LANES,), jnp.float32),
            pltpu.SemaphoreType.DMA,
            pltpu.SemaphoreType.DMA,
        ],
    )
    return fn(packed, h)


def kernel(h, A, W, b, bag_label):
    a_i = A[:, 0, bag_label]
    packed = jnp.concatenate(
        [a_i, b, jnp.zeros((6,), jnp.float32), W.T.reshape(2 * D)])
    probs_flat = _sc_call(packed, h)
    logits = probs_flat.reshape(2, 2 * N_INS).T.reshape(2 * N_INS, 1, 2)
    labels = jnp.concatenate([jnp.ones((N_INS,), jnp.int32),
                              jnp.zeros((N_INS,), jnp.int32)])
    return labels, logits


# speedup vs baseline: 1.0120x; 1.0120x over previous
import jax
import jax.numpy as jnp
from jax import lax
from jax.experimental import pallas as pl
from jax.experimental.pallas import tpu as pltpu
from jax.experimental.pallas import tpu_sc as plsc

N = 20000
D = 512
N_INS = 8
LANES = 16
NTILES = 16
CHUNK = 1280
LAST_BASE = N - CHUNK
LAST_LO = (NTILES - 1) * CHUNK
INNER = 4
OUTER = CHUNK // (LANES * INNER)
BIG_I = 2 ** 30
SENT = -2.0
MASKED = -3.0


def _insert_step(v, gi, ms, mi):
    c = [v > m for m in ms]
    nm = [jnp.where(c[0], v, ms[0])]
    ni = [jnp.where(c[0], gi, mi[0])]
    for k in range(1, N_INS):
        nm.append(jnp.where(c[k], jnp.where(c[k - 1], ms[k - 1], v), ms[k]))
        ni.append(jnp.where(c[k], jnp.where(c[k - 1], mi[k - 1], gi), mi[k]))
    return nm, ni


def _extract8(ms, mi, iota):
    def round_body(r, st):
        selv, seli, pv, pi = st
        m = jnp.full((LANES,), SENT, jnp.float32)
        ii = jnp.full((LANES,), BIG_I, jnp.int32)
        for k in range(N_INS):
            v, gi = ms[k], mi[k]
            elig = (v < pv) | ((v == pv) & (gi > pi))
            veff = jnp.where(elig, v, jnp.float32(SENT))
            upd = veff > m
            m = jnp.where(upd, veff, m)
            ii = jnp.where(upd, gi, ii)
        mval = jnp.max(m)
        midx = jnp.min(jnp.where(m == mval, ii, BIG_I))
        selv = jnp.where(iota == r, mval, selv)
        seli = jnp.where(iota == r, midx, seli)
        return (selv, seli, mval, midx)

    st0 = (jnp.full((LANES,), SENT, jnp.float32),
           jnp.zeros((LANES,), jnp.int32),
           jnp.float32(2.0), jnp.int32(-1))
    selv, seli, _, _ = lax.fori_loop(0, N_INS, round_body, st0)
    return selv, seli


def _fresh_state():
    return ([jnp.full((LANES,), SENT, jnp.float32) for _ in range(N_INS)],
            [jnp.full((LANES,), BIG_I, jnp.int32) for _ in range(N_INS)])


WB_OFF = N
WB_LEN = 8 + 2 * D
W_OFF = 8


def _sc_body(pk_hbm, h_hbm, probs_hbm, lab_hbm,
             a_v, st_f, st_i, spm_f, spm_i, cand_f, cand_i,
             idx_v, rows_v, wb_v, o0_v, o1_v, lab_v, sem, sem2):
    cid = lax.axis_index("c")
    sid = lax.axis_index("s")
    iota = lax.iota(jnp.int32, LANES)
    last = sid == NTILES - 1
    base = jnp.where(last, LAST_BASE, sid * CHUNK)
    lo = jnp.where(last, LAST_LO, 0)

    copy_a = pltpu.async_copy(pk_hbm.at[pl.ds(base, CHUNK)], a_v, sem)

    @pl.when(sid == 0)
    def _():
        pltpu.async_copy(pk_hbm.at[pl.ds(WB_OFF, WB_LEN)], wb_v, sem2)

    copy_a.wait()

    sgn = jnp.where(cid == 0, jnp.float32(1.0), jnp.float32(-1.0))
    bi = base + iota

    def scan_body(j, st):
        ms, mi = list(st[0]), list(st[1])
        off0 = j * (LANES * INNER)
        for k in range(INNER):
            off = off0 + k * LANES
            gi = bi + off
            v = a_v[pl.ds(off, LANES)] * sgn
            v = jnp.where(gi >= lo, v, jnp.float32(MASKED))
            ms, mi = _insert_step(v, gi, ms, mi)
        return (tuple(ms), tuple(mi))

    ms0, mi0 = _fresh_state()
    ms, mi = lax.fori_loop(0, OUTER, scan_body, (tuple(ms0), tuple(mi0)))
    selv, seli = _extract8(list(ms), list(mi), iota)

    st_f[...] = selv
    st_i[...] = seli
    pltpu.sync_copy(st_f, spm_f.at[pl.ds(sid * LANES, LANES)])
    pltpu.sync_copy(st_i, spm_i.at[pl.ds(sid * LANES, LANES)])
    plsc.subcore_barrier()

    @pl.when(sid == 0)
    def _():
        pltpu.sync_copy(spm_f, cand_f)
        pltpu.sync_copy(spm_i, cand_i)

        def merge_body(t, st):
            ms, mi = list(st[0]), list(st[1])
            v = cand_f[pl.ds(t * LANES, LANES)]
            gi = cand_i[pl.ds(t * LANES, LANES)]
            ms, mi = _insert_step(v, gi, ms, mi)
            return (tuple(ms), tuple(mi))

        ms0, mi0 = _fresh_state()
        ms, mi = lax.fori_loop(0, NTILES, merge_body, (tuple(ms0), tuple(mi0)))
        _, gsel = _extract8(list(ms), list(mi), iota)

        idx_v[...] = jnp.where(iota < N_INS, gsel, 0)
        pltpu.async_copy(h_hbm.at[idx_v], rows_v, sem).wait()

        pltpu.make_async_copy(pk_hbm.at[pl.ds(WB_OFF, WB_LEN)], wb_v,
                              sem2).wait()

        def mm_body(j, st):
            a0s, a1s = list(st[0]), list(st[1])
            wt0 = wb_v[pl.ds(W_OFF + j * LANES, LANES)]
            wt1 = wb_v[pl.ds(W_OFF + D + j * LANES, LANES)]
            for i in range(N_INS):
                rv = rows_v[i, 0, pl.ds(j * LANES, LANES)]
                a0s[i] = a0s[i] + rv * wt0
                a1s[i] = a1s[i] + rv * wt1
            return (tuple(a0s), tuple(a1s))

        z = [jnp.zeros((LANES,), jnp.float32) for _ in range(N_INS)]
        a0s, a1s = lax.fori_loop(0, D // LANES, mm_body,
                                 (tuple(z), tuple(z)))
        l0 = jnp.zeros((LANES,), jnp.float32)
        l1 = jnp.zeros((LANES,), jnp.float32)
        for i in range(N_INS):
            l0 = jnp.where(iota == i, jnp.sum(a0s[i]), l0)
            l1 = jnp.where(iota == i, jnp.sum(a1s[i]), l1)

        bv = wb_v[pl.ds(0, LANES)]
        l0 = l0 + bv[0]
        l1 = l1 + bv[1]
        o0_v[...] = 1.0 / (1.0 + jnp.exp(l1 - l0))
        o1_v[...] = 1.0 / (1.0 + jnp.exp(l0 - l1))
        pltpu.sync_copy(o0_v.at[pl.ds(0, N_INS)],
                        probs_hbm.at[pl.ds(cid * N_INS, N_INS)])
        pltpu.sync_copy(o1_v.at[pl.ds(0, N_INS)],
                        probs_hbm.at[pl.ds(2 * N_INS + cid * N_INS, N_INS)])

        lab_v[...] = jnp.broadcast_to(1 - cid, (LANES,)).astype(jnp.int32)
        pltpu.sync_copy(lab_v.at[pl.ds(0, N_INS)],
                        lab_hbm.at[pl.ds(cid * N_INS, N_INS)])


@jax.jit
def _sc_call(packed, h):
    mesh = plsc.VectorSubcoreMesh(core_axis_name="c", subcore_axis_name="s")
    fn = pl.kernel(
        _sc_body,
        mesh=mesh,
        out_type=[jax.ShapeDtypeStruct((4 * N_INS,), jnp.float32),
                  jax.ShapeDtypeStruct((2 * N_INS,), jnp.int32)],
        compiler_params=pltpu.CompilerParams(
            needs_layout_passes=False, use_tc_tiling_on_sc=False,
            disable_bounds_checks=True, disable_semaphore_checks=True),
        scratch_types=[
            pltpu.VMEM((CHUNK,), jnp.float32),
            pltpu.VMEM((LANES,), jnp.float32),
            pltpu.VMEM((LANES,), jnp.int32),
            pltpu.VMEM_SHARED((NTILES * LANES,), jnp.float32),
            pltpu.VMEM_SHARED((NTILES * LANES,), jnp.int32),
            pltpu.VMEM((NTILES * LANES,), jnp.float32),
            pltpu.VMEM((NTILES * LANES,), jnp.int32),
            pltpu.VMEM((LANES,), jnp.int32),
            pltpu.VMEM((LANES, 1, D), jnp.float32),
            pltpu.VMEM((8 + 2 * D,), jnp.float32),
            pltpu.VMEM((LANES,), jnp.float32),
            pltpu.VMEM((LANES,), jnp.float32),
            pltpu.VMEM((LANES,), jnp.int32),
            pltpu.SemaphoreType.DMA,
            pltpu.SemaphoreType.DMA,
        ],
    )
    return fn(packed, h)


def kernel(h, A, W, b, bag_label):
    a_i = A[:, 0, bag_label]
    packed = jnp.concatenate(
        [a_i, b, jnp.zeros((6,), jnp.float32), W.T.reshape(2 * D)])
    probs_flat, labels = _sc_call(packed, h)
    logits = probs_flat.reshape(2, 2 * N_INS).T.reshape(2 * N_INS, 1, 2)
    return labels, logits
